# Initial kernel scaffold; baseline (speedup 1.0000x reference)
import jax
import jax.numpy as jnp
from jax.experimental import pallas as pl


def kernel(tokens, unk, weights):
    raise NotImplementedError("write your pallas kernel here")


# speedup vs baseline: 4.4783x; 4.4783x over previous
import functools

import jax
import jax.numpy as jnp
from jax import lax
from jax.experimental import pallas as pl
from jax.experimental.pallas import tpu as pltpu
from jax.experimental.pallas import tpu_sc as plsc

EMBED_DIM = 64
N_TOKENS = 4096 * 50
NC, NS, L = 2, 16, 16
NW = NC * NS
ROWS_PER_W = N_TOKENS // NW
SB = 128
K = ROWS_PER_W // SB
G = SB // L


def _sc_body(tok_hbm, unk_hbm, w_hbm, out_hbm, idx_raw, idx_adj, unk_v,
             rows, sem):
    wid = lax.axis_index("s") * NC + lax.axis_index("c")
    pltpu.sync_copy(tok_hbm.at[wid], idx_raw)
    pltpu.sync_copy(unk_hbm, unk_v)
    out_base = wid * ROWS_PER_W

    def subbatch(k, _):
        bad = jnp.zeros((L,), jnp.bool_)
        for o in range(G):
            v = idx_raw[k, pl.ds(o * L, L)]
            idx_adj[k, pl.ds(o * L, L)] = jnp.maximum(v - 2, 0)
            bad = jnp.logical_or(bad, v < 2)
        nbad = plsc.all_reduce_population_count(bad)
        allok = nbad[0] == 0
        pltpu.async_copy(w_hbm.at[idx_adj.at[k]], rows, sem).wait()

        def fixup():
            for o in range(G):
                v = idx_raw[k, pl.ds(o * L, L)]
                msk = v < 2
                f = v.astype(jnp.float32)
                rowvec = o * L + lax.iota(jnp.int32, L)

                def colbody(c, _):
                    cvec = jnp.full((L,), c, jnp.int32)
                    unkc = plsc.load_gather(unk_v, [cvec])
                    plsc.store_scatter(rows, [rowvec, cvec], unkc * f,
                                       mask=msk)
                    return 0

                lax.fori_loop(0, EMBED_DIM, colbody, 0)

        lax.cond(allok, lambda: None, fixup)
        pltpu.sync_copy(rows, out_hbm.at[pl.ds(out_base + k * SB, SB)])
        return 0

    lax.fori_loop(0, K, subbatch, 0)


@jax.jit
def kernel(tokens, unk, weights):
    tok3 = tokens.reshape(NW, K, SB).astype(jnp.int32)
    unk1 = unk.reshape(EMBED_DIM)
    mesh = plsc.VectorSubcoreMesh(core_axis_name="c", subcore_axis_name="s")
    out = pl.kernel(
        _sc_body,
        out_type=jax.ShapeDtypeStruct((N_TOKENS, EMBED_DIM), jnp.float32),
        mesh=mesh,
        compiler_params=pltpu.CompilerParams(needs_layout_passes=False,
                                             use_tc_tiling_on_sc=False),
        scratch_types=[
            pltpu.VMEM((K, SB), jnp.int32),
            pltpu.VMEM((K, SB), jnp.int32),
            pltpu.VMEM((EMBED_DIM,), jnp.float32),
            pltpu.VMEM((SB, EMBED_DIM), jnp.float32),
            pltpu.SemaphoreType.DMA,
        ],
    )(tok3, unk1, weights)
    return out.reshape(4096, 50, EMBED_DIM)


# speedup vs baseline: 5.1228x; 1.1439x over previous
import jax
import jax.numpy as jnp
from jax import lax
from jax.experimental import pallas as pl
from jax.experimental.pallas import tpu as pltpu
from jax.experimental.pallas import tpu_sc as plsc

EMBED_DIM = 64
N_TOKENS = 4096 * 50
NC, NS, L = 2, 16, 16
NW = NC * NS
ROWS_PER_W = N_TOKENS // NW
SB = 128
K = ROWS_PER_W // SB
G = SB // L
NBUF = 5


def _sc_body(tok_hbm, unk_hbm, w_hbm, out_hbm, idx_raw, idx_adj, unk_v,
             rows, gsem, ssem):
    wid = lax.axis_index("s") * NC + lax.axis_index("c")
    pltpu.sync_copy(tok_hbm.at[wid], idx_raw)
    pltpu.sync_copy(unk_hbm, unk_v)
    out_base = wid * ROWS_PER_W

    def adjust(k, _):
        for o in range(G):
            v = idx_raw[k, pl.ds(o * L, L)]
            idx_adj[k, pl.ds(o * L, L)] = jnp.maximum(v - 2, 0)
        return 0

    def sg(k, b):
        pltpu.async_copy(w_hbm.at[idx_adj.at[k]], rows.at[b], gsem.at[b])

    def wg(k, b):
        pltpu.make_async_copy(w_hbm.at[idx_adj.at[k]], rows.at[b],
                              gsem.at[b]).wait()

    def out_slice(k):
        return out_hbm.at[pl.ds(out_base + k * SB, SB)]

    def ss(k, b):
        pltpu.async_copy(rows.at[b], out_slice(k), ssem.at[b])

    def ws(k, b):
        pltpu.make_async_copy(rows.at[b], out_slice(k), ssem.at[b]).wait()

    def fix(k, b):
        bad = jnp.zeros((L,), jnp.bool_)
        for o in range(G):
            v = idx_raw[k, pl.ds(o * L, L)]
            bad = jnp.logical_or(bad, v < 2)
        nbad = plsc.all_reduce_population_count(bad)

        def fixup():
            for o in range(G):
                v = idx_raw[k, pl.ds(o * L, L)]
                msk = v < 2
                f = v.astype(jnp.float32)
                rowvec = o * L + lax.iota(jnp.int32, L)

                def colbody(c, _):
                    cvec = jnp.full((L,), c, jnp.int32)
                    unkc = plsc.load_gather(unk_v, [cvec])
                    plsc.store_scatter(rows.at[b], [rowvec, cvec], unkc * f,
                                       mask=msk)
                    return 0

                lax.fori_loop(0, EMBED_DIM, colbody, 0)

        lax.cond(nbad[0] == 0, lambda: None, fixup)

    lax.fori_loop(0, NBUF - 1, adjust, 0)
    for b in range(NBUF - 1):
        sg(b, b)
    lax.fori_loop(NBUF - 1, K, adjust, 0)

    wg(0, 0)
    fix(0, 0)
    ss(0, 0)
    sg(NBUF - 1, NBUF - 1)

    def pos(j, b):
        wg(j, b)
        fix(j, b)
        ss(j, b)
        ws(j - 1, (b - 1) % NBUF)
        sg(j + NBUF - 1, (b + NBUF - 1) % NBUF)

    def round_body(rr, _):
        for b0 in range(NBUF):
            pos(1 + rr * NBUF + b0, (1 + b0) % NBUF)
        return 0

    lax.fori_loop(0, (K - NBUF) // NBUF, round_body, 0)

    for j in range(K - NBUF + 1, K):
        wg(j, j % NBUF)
        fix(j, j % NBUF)
        ss(j, j % NBUF)
    for j in range(K - NBUF, K):
        ws(j, j % NBUF)


@jax.jit
def kernel(tokens, unk, weights):
    tok3 = tokens.reshape(NW, K, SB).astype(jnp.int32)
    unk1 = unk.reshape(EMBED_DIM)
    mesh = plsc.VectorSubcoreMesh(core_axis_name="c", subcore_axis_name="s")
    out = pl.kernel(
        _sc_body,
        out_type=jax.ShapeDtypeStruct((N_TOKENS, EMBED_DIM), jnp.float32),
        mesh=mesh,
        compiler_params=pltpu.CompilerParams(needs_layout_passes=False,
                                             use_tc_tiling_on_sc=False),
        scratch_types=[
            pltpu.VMEM((K, SB), jnp.int32),
            pltpu.VMEM((K, SB), jnp.int32),
            pltpu.VMEM((EMBED_DIM,), jnp.float32),
            pltpu.VMEM((NBUF, SB, EMBED_DIM), jnp.float32),
            pltpu.SemaphoreType.DMA((NBUF,)),
            pltpu.SemaphoreType.DMA((NBUF,)),
        ],
    )(tok3, unk1, weights)
    return out.reshape(4096, 50, EMBED_DIM)


# speedup vs baseline: 5.1237x; 1.0002x over previous
import jax
import jax.numpy as jnp
from jax import lax
from jax.experimental import pallas as pl
from jax.experimental.pallas import tpu as pltpu
from jax.experimental.pallas import tpu_sc as plsc

EMBED_DIM = 64
SEQ = 50
BATCH = 4096
N_TOKENS = BATCH * SEQ
NC, NS, L = 2, 16, 16
NW = NC * NS
K = BATCH // NW
NBUF = 8
SLICES = (0, 16, 32, SEQ - L)


def _sc_body(tok_hbm, unk_hbm, w_hbm, out_hbm, idx_raw, idx_adj, unk_v,
             rows, gsem, ssem):
    wid = lax.axis_index("s") * NC + lax.axis_index("c")
    tr_base = wid * K
    pltpu.sync_copy(tok_hbm.at[wid], idx_raw)
    pltpu.sync_copy(unk_hbm, unk_v)

    def adjust(k, _):
        for lo in SLICES:
            v = idx_raw[k, pl.ds(lo, L)]
            idx_adj[k, pl.ds(lo, L)] = jnp.maximum(v - 2, 0)
        return 0

    def sg(k, b):
        pltpu.async_copy(w_hbm.at[idx_adj.at[k]], rows.at[b], gsem.at[b])

    def wg(k, b):
        pltpu.make_async_copy(w_hbm.at[idx_adj.at[k]], rows.at[b],
                              gsem.at[b]).wait()

    def ss(k, b):
        pltpu.async_copy(rows.at[b], out_hbm.at[tr_base + k], ssem.at[b])

    def ws(k, b):
        pltpu.make_async_copy(rows.at[b], out_hbm.at[tr_base + k],
                              ssem.at[b]).wait()

    def fix(k, b):
        bad = jnp.zeros((L,), jnp.bool_)
        for lo in SLICES:
            v = idx_raw[k, pl.ds(lo, L)]
            bad = jnp.logical_or(bad, v < 2)
        nbad = plsc.all_reduce_population_count(bad)

        def fixup():
            for lo in SLICES:
                v = idx_raw[k, pl.ds(lo, L)]
                msk = v < 2
                f = v.astype(jnp.float32)
                seqvec = lo + lax.iota(jnp.int32, L)

                def colbody(c, _):
                    cvec = jnp.full((L,), c, jnp.int32)
                    unkc = plsc.load_gather(unk_v, [cvec])
                    plsc.store_scatter(rows.at[b], [seqvec, cvec], unkc * f,
                                       mask=msk)
                    return 0

                lax.fori_loop(0, EMBED_DIM, colbody, 0)

        lax.cond(nbad[0] == 0, lambda: None, fixup)

    lax.fori_loop(0, NBUF - 1, adjust, 0)
    for b in range(NBUF - 1):
        sg(b, b)
    lax.fori_loop(NBUF - 1, K, adjust, 0)

    wg(0, 0)
    fix(0, 0)
    ss(0, 0)
    sg(NBUF - 1, NBUF - 1)

    def pos(j, b):
        wg(j, b)
        fix(j, b)
        ss(j, b)
        ws(j - 1, (b - 1) % NBUF)
        sg(j + NBUF - 1, (b + NBUF - 1) % NBUF)

    def round_body(rr, _):
        for b0 in range(NBUF):
            pos(1 + rr * NBUF + b0, (1 + b0) % NBUF)
        return 0

    lax.fori_loop(0, (K - NBUF) // NBUF, round_body, 0)

    for j in range(K - NBUF + 1, K):
        wg(j, j % NBUF)
        fix(j, j % NBUF)
        ss(j, j % NBUF)
    for j in range(K - NBUF, K):
        ws(j, j % NBUF)


@jax.jit
def kernel(tokens, unk, weights):
    tok3 = tokens.reshape(NW, K, SEQ).astype(jnp.int32)
    unk1 = unk.reshape(EMBED_DIM)
    mesh = plsc.VectorSubcoreMesh(core_axis_name="c", subcore_axis_name="s")
    out = pl.kernel(
        _sc_body,
        out_type=jax.ShapeDtypeStruct((BATCH, SEQ, EMBED_DIM), jnp.float32),
        mesh=mesh,
        compiler_params=pltpu.CompilerParams(needs_layout_passes=False,
                                             use_tc_tiling_on_sc=False),
        scratch_types=[
            pltpu.VMEM((K, SEQ), jnp.int32),
            pltpu.VMEM((K, SEQ), jnp.int32),
            pltpu.VMEM((EMBED_DIM,), jnp.float32),
            pltpu.VMEM((NBUF, SEQ, EMBED_DIM), jnp.float32),
            pltpu.SemaphoreType.DMA((NBUF,)),
            pltpu.SemaphoreType.DMA((NBUF,)),
        ],
    )(tok3, unk1, weights)
    return out
